# baseline (speedup 1.0000x reference)
import jax
import jax.numpy as jnp
from jax.experimental import pallas as pl
from jax.experimental.pallas import tpu as pltpu

_NS = 5
_BETA = 0.75
_EPS = 1e-9
_ROWS = 16384
_D = 64
_R2 = _ROWS // 2
_BLK = 1024
_GRID = _R2 // _BLK


def _tc_body(idx_ref, ctx_ref, tgt_ref, emb_ref, out_ref, etmp, erows, sem):
    step = pl.program_id(0)

    @pl.when(step == 0)
    def _init():
        out_ref[0, 0] = 0.0
        etmp[...] = jnp.zeros_like(etmp)
        for s in range(_NS):
            cp = pltpu.make_async_copy(
                emb_ref.at[pl.ds(idx_ref[s], 1)], etmp.at[pl.ds(s, 1)], sem)
            cp.start()
            cp.wait()
        ev = etmp[...]
        z = jnp.zeros_like(ev)
        erows[...] = jnp.concatenate(
            [jnp.concatenate([ev, z], axis=1),
             jnp.concatenate([z, ev], axis=1)], axis=0)

    c = ctx_ref[...]
    t = tgt_ref[...]
    p = c * t

    ri = jax.lax.broadcasted_iota(jnp.int32, (8, 2 * _D), 0)
    li = jax.lax.broadcasted_iota(jnp.int32, (8, 2 * _D), 1)
    m = jnp.where(((ri == 0) & (li < _D)) | ((ri == 1) & (li >= _D)), 1.0, 0.0)
    td = jax.lax.dot_general(m, p, (((1,), (1,)), ((), ())),
                             preferred_element_type=jnp.float32)
    lt = jnp.log(jnp.clip(1.0 / (1.0 + jnp.exp(-td)), _EPS, None))
    tri = jax.lax.broadcasted_iota(jnp.int32, lt.shape, 0)
    lt = jnp.where(tri < 2, lt, 0.0)

    e = erows[...]
    sd = jax.lax.dot_general(e, c, (((1,), (1,)), ((), ())),
                             preferred_element_type=jnp.float32)
    ls = jnp.log(jnp.clip(1.0 / (1.0 + jnp.exp(sd)), _BETA, None))
    sri = jax.lax.broadcasted_iota(jnp.int32, ls.shape, 0)
    ls = jnp.where((sri % 8) < _NS, ls, 0.0)

    out_ref[0, 0] += jnp.sum(lt) + jnp.sum(ls)


def kernel(context, target, emb_table, sample_indices):
    c2 = context.reshape(_R2, 2 * _D)
    t2 = target.reshape(_R2, 2 * _D)
    grid_spec = pltpu.PrefetchScalarGridSpec(
        num_scalar_prefetch=1,
        grid=(_GRID,),
        in_specs=[
            pl.BlockSpec((_BLK, 2 * _D), lambda i, idx: (i, 0)),
            pl.BlockSpec((_BLK, 2 * _D), lambda i, idx: (i, 0)),
            pl.BlockSpec(memory_space=pltpu.MemorySpace.HBM),
        ],
        out_specs=pl.BlockSpec(memory_space=pltpu.MemorySpace.SMEM),
        scratch_shapes=[
            pltpu.VMEM((8, _D), jnp.float32),
            pltpu.VMEM((16, 2 * _D), jnp.float32),
            pltpu.SemaphoreType.DMA,
        ],
    )
    out = pl.pallas_call(
        _tc_body,
        grid_spec=grid_spec,
        out_shape=jax.ShapeDtypeStruct((1, 1), jnp.float32),
    )(sample_indices.astype(jnp.int32), c2, t2, emb_table)
    return out[0, 0]


# speedup vs baseline: 1.2119x; 1.2119x over previous
import jax
import jax.numpy as jnp
from jax.experimental import pallas as pl
from jax.experimental.pallas import tpu as pltpu

_NS = 5
_BETA = 0.75
_EPS = 1e-9
_ROWS = 16384
_D = 64
_BLK = 2048
_GRID = _ROWS // _BLK


def _tc_body(idx_ref, ctx_ref, tgt_ref, emb_ref, out_ref, erows, sem):
    step = pl.program_id(0)

    @pl.when(step == 0)
    def _init():
        out_ref[0, 0] = 0.0
        erows[...] = jnp.zeros_like(erows)
        for s in range(_NS):
            cp = pltpu.make_async_copy(
                emb_ref.at[pl.ds(idx_ref[s], 1)], erows.at[pl.ds(s, 1)], sem)
            cp.start()
            cp.wait()

    c = ctx_ref[...]
    t = tgt_ref[...]
    p = c * t

    ri = jax.lax.broadcasted_iota(jnp.int32, (8, _D), 0)
    m = jnp.where(ri == 0, 1.0, 0.0)
    td = jax.lax.dot_general(m, p, (((1,), (1,)), ((), ())),
                             preferred_element_type=jnp.float32)
    lt = jnp.log(jnp.clip(1.0 / (1.0 + jnp.exp(-td)), _EPS, None))
    tri = jax.lax.broadcasted_iota(jnp.int32, lt.shape, 0)
    lt = jnp.where(tri == 0, lt, 0.0)

    e = erows[...]
    sd = jax.lax.dot_general(e, c, (((1,), (1,)), ((), ())),
                             preferred_element_type=jnp.float32)
    ls = jnp.log(jnp.clip(1.0 / (1.0 + jnp.exp(sd)), _BETA, None))
    sri = jax.lax.broadcasted_iota(jnp.int32, ls.shape, 0)
    ls = jnp.where(sri < _NS, ls, 0.0)

    out_ref[0, 0] += jnp.sum(lt) + jnp.sum(ls)


def kernel(context, target, emb_table, sample_indices):
    grid_spec = pltpu.PrefetchScalarGridSpec(
        num_scalar_prefetch=1,
        grid=(_GRID,),
        in_specs=[
            pl.BlockSpec((_BLK, _D), lambda i, idx: (i, 0)),
            pl.BlockSpec((_BLK, _D), lambda i, idx: (i, 0)),
            pl.BlockSpec(memory_space=pltpu.MemorySpace.HBM),
        ],
        out_specs=pl.BlockSpec(memory_space=pltpu.MemorySpace.SMEM),
        scratch_shapes=[
            pltpu.VMEM((8, _D), jnp.float32),
            pltpu.SemaphoreType.DMA,
        ],
    )
    out = pl.pallas_call(
        _tc_body,
        grid_spec=grid_spec,
        out_shape=jax.ShapeDtypeStruct((1, 1), jnp.float32),
    )(sample_indices.astype(jnp.int32), context, target, emb_table)
    return out[0, 0]
